# baseline (speedup 1.0000x reference)
import functools

import jax
import jax.numpy as jnp
from jax import lax
from jax.experimental import pallas as pl
from jax.experimental.pallas import tpu as pltpu
from jax.experimental.pallas import tpu_sc as plsc

_N = 10000
_E = 320000
_BN = 128
_F32 = jnp.float32

_NC = 2
_NS = 16
_NW = _NC * _NS
_CHUNK = 64
_NCH_A = 156
_NCH_B = 157
_NTA = 24
_EPT_A = _NCH_A * _CHUNK
_EPT_B = _NCH_B * _CHUNK
_SPLIT = _NTA * _EPT_A
_NCHP = 160
_PG = 16
_SROWS = _N
_ZR = 8
_ZW = 79
_CP0 = 624


def _lk(x):
    return jnp.maximum(x, x * 0.01)


def _comb_body(ws, wd, we, be, bs, bd, wt1, bt1, wst_o, wdt_o, wet_o, cvec_o):
    wst_o[...] = jnp.dot(ws[...], wt1[...], preferred_element_type=_F32)
    wdt_o[...] = jnp.dot(wd[...], wt1[...], preferred_element_type=_F32)
    wet_o[...] = jnp.dot(we[...], wt1[...], preferred_element_type=_F32)
    bsum = be[...] + bs[...] + bd[...]
    cvec_o[...] = jnp.dot(bsum, wt1[...], preferred_element_type=_F32) + bt1[...]


def _ab_body(x, wst, wdt, a_o, b_o):
    xv = x[...]
    a_o[...] = jnp.dot(xv, wst[...], preferred_element_type=_F32)
    b_o[...] = jnp.dot(xv, wdt[...], preferred_element_type=_F32)


def _c_body(ea, wet, cvec, c_o):
    c_o[...] = jnp.dot(ea[...], wet[...], preferred_element_type=_F32) + cvec[...]


def _sc_body(src_h, dst_h, sidx_h, a_hbm, b_hbm, c_hbm, z128,
             s_out,
             src2d, dst2d, sidx2d, a_v, b_v, c_v, s_sh,
             sema, semb, semc):
    cid = lax.axis_index("c")
    sid = lax.axis_index("s")
    wid = sid * _NC + cid

    zrow = sid * (_N // _NS)

    def _zs(j, c):
        zo = jnp.minimum(zrow + j * _ZR, _N - _ZR)
        pltpu.sync_copy(z128, s_sh.at[pl.ds(zo, _ZR)])
        return c

    lax.fori_loop(0, _ZW, _zs, 0)
    plsc.subcore_barrier()

    ebase = jnp.where(wid < _NTA, wid * _EPT_A,
                      _SPLIT + (wid - _NTA) * _EPT_B)
    nch = jnp.where(wid < _NTA, _NCH_A, _NCH_B)

    def page(p, carry0):
        pbase = p * _PG

        def _ld(j, c):
            sl8 = pl.ds(j * 8, 8)
            slh = pl.ds(pbase + j * 8, 8)
            pltpu.sync_copy(src_h.at[wid, slh], src2d.at[sl8])
            pltpu.sync_copy(dst_h.at[wid, slh], dst2d.at[sl8])
            pltpu.sync_copy(sidx_h.at[wid, slh], sidx2d.at[sl8])
            return c

        lax.fori_loop(0, _PG // 8, _ld, 0)
        cnt = jnp.minimum(nch - pbase, _PG)

        def chunk(i, carry):
            it = pbase + i
            base = ebase + it * _CHUNK
            cpa = pltpu.async_copy(a_hbm.at[src2d.at[i]], a_v, sema)
            cpb = pltpu.async_copy(b_hbm.at[dst2d.at[i]], b_v, semb)
            cpc = pltpu.async_copy(c_hbm.at[pl.ds(base, _CHUNK)], c_v, semc)
            cpa.wait()
            cpb.wait()
            cpc.wait()

            def row(r, c2):
                for k in range(_BN // 16):
                    sl = pl.ds(k * 16, 16)
                    z = a_v[r, sl] + b_v[r, sl] + c_v[r, sl]
                    a_v[r, sl] = jnp.maximum(z, z * 0.01)
                return c2

            lax.fori_loop(0, _CHUNK, row, 0)
            pltpu.sync_copy(a_v, s_sh.at[sidx2d.at[i]], add=True)
            return carry

        lax.fori_loop(0, cnt, chunk, 0)
        return carry0

    lax.fori_loop(0, _NCHP // _PG, page, 0)
    plsc.subcore_barrier()

    off = sid * _CP0

    @pl.when(sid < _NS - 1)
    def _copy_main():
        pltpu.sync_copy(s_sh.at[pl.ds(off, _CP0)],
                        s_out.at[cid, pl.ds(off, _CP0)])

    @pl.when(sid == _NS - 1)
    def _copy_last():
        lo = (_NS - 1) * _CP0
        cp1 = _N - lo
        pltpu.sync_copy(s_sh.at[pl.ds(lo, cp1)], s_out.at[cid, pl.ds(lo, cp1)])


def _deg_body(dst_h, zdeg, deg_out,
              dst2d, ones_v, deg_sh, semd):
    cid = lax.axis_index("c")
    sid = lax.axis_index("s")
    wid = sid * _NC + cid

    def _ones_row(r, c):
        ones_v[r, :] = jnp.full((16,), 1.0, _F32)
        return c

    lax.fori_loop(0, _CHUNK, _ones_row, 0)

    zrow = sid * (_N // _NS)

    def _zs(j, c):
        zo = jnp.minimum(zrow + j * _ZR, _N - _ZR)
        pltpu.sync_copy(zdeg, deg_sh.at[pl.ds(zo, _ZR)])
        return c

    lax.fori_loop(0, _ZW, _zs, 0)
    plsc.subcore_barrier()

    nch = jnp.where(wid < _NTA, _NCH_A, _NCH_B)

    def page(p, carry0):
        pbase = p * _PG

        def _ld(j, c):
            pltpu.sync_copy(dst_h.at[wid, pl.ds(pbase + j * 8, 8)],
                            dst2d.at[pl.ds(j * 8, 8)])
            return c

        lax.fori_loop(0, _PG // 8, _ld, 0)
        cnt = jnp.minimum(nch - pbase, _PG)

        def chunk(i, carry):
            pltpu.sync_copy(ones_v, deg_sh.at[dst2d.at[i]], add=True)
            return carry

        lax.fori_loop(0, cnt, chunk, 0)
        return carry0

    lax.fori_loop(0, _NCHP // _PG, page, 0)
    plsc.subcore_barrier()

    off = sid * _CP0

    @pl.when(sid < _NS - 1)
    def _copy_main():
        pltpu.sync_copy(deg_sh.at[pl.ds(off, _CP0)],
                        deg_out.at[cid, pl.ds(off, _CP0)])

    @pl.when(sid == _NS - 1)
    def _copy_last():
        lo = (_NS - 1) * _CP0
        cp1 = _N - lo
        pltpu.sync_copy(deg_sh.at[pl.ds(lo, cp1)],
                        deg_out.at[cid, pl.ds(lo, cp1)])


def _out_body(x, sp, dp, wt2, bt2, wpd, bpd, wpe, bpe,
              wp1, bp1, wp2, bp2, wp3, bp3, o):
    s = sp[0] + sp[1]
    dsum = dp[0] + dp[1]
    deg = dsum[:, 0:1]
    msg = jnp.dot(s, wt2[...], preferred_element_type=_F32) + deg * bt2[...]
    h = (jnp.dot(x[...], wpd[...], preferred_element_type=_F32) + bpd[...]
         + jnp.dot(msg, wpe[...], preferred_element_type=_F32) + bpe[...])
    h = _lk(h)
    h = _lk(jnp.dot(h, wp1[...], preferred_element_type=_F32) + bp1[...])
    h = _lk(jnp.dot(h, wp2[...], preferred_element_type=_F32) + bp2[...])
    o[...] = jnp.dot(h, wp3[...], preferred_element_type=_F32) + bp3[...]


def _full(shape):
    return pl.BlockSpec(shape, lambda i: tuple(0 for _ in shape))


def kernel(node_feat, edge_index, edge_attr, We, be, Ws, bs, Wd, bd,
           Wt1, bt1, Wt2, bt2, Wpd, bpd, Wpe, bpe,
           Wp1, bp1, Wp2, bp2, Wp3, bp3):
    f32 = _F32
    be2, bs2, bd2, bt12 = (b.reshape(1, -1) for b in (be, bs, bd, bt1))
    bt22, bpd2, bpe2 = (b.reshape(1, -1) for b in (bt2, bpd, bpe))
    bp12, bp22, bp32 = (b.reshape(1, -1) for b in (bp1, bp2, bp3))

    wst, wdt, wet, cvec = pl.pallas_call(
        _comb_body,
        out_shape=[jax.ShapeDtypeStruct((128, 128), f32),
                   jax.ShapeDtypeStruct((128, 128), f32),
                   jax.ShapeDtypeStruct((16, 128), f32),
                   jax.ShapeDtypeStruct((1, 128), f32)],
    )(Ws, Wd, We, be2, bs2, bd2, Wt1, bt12)

    rn = 1000
    hb = lambda i: (i, 0)
    a_arr, b_arr = pl.pallas_call(
        _ab_body,
        grid=(_N // rn,),
        in_specs=[pl.BlockSpec((rn, 128), hb),
                  _full((128, 128)), _full((128, 128))],
        out_specs=[pl.BlockSpec((rn, 128), hb)] * 2,
        out_shape=[jax.ShapeDtypeStruct((_N, _BN), f32)] * 2,
    )(node_feat, wst, wdt)

    pad_a = ((0, 0), (0, _NCHP - _NCH_A), (0, 0))
    pad_b = ((0, 0), (0, _NCHP - _NCH_B), (0, 0))
    src_h = jnp.concatenate(
        [jnp.pad(edge_index[0, :_SPLIT].reshape(_NTA, _NCH_A, _CHUNK), pad_a),
         jnp.pad(edge_index[0, _SPLIT:].reshape(_NW - _NTA, _NCH_B, _CHUNK),
                 pad_b)], axis=0)
    dst_h = jnp.concatenate(
        [jnp.pad(edge_index[1, :_SPLIT].reshape(_NTA, _NCH_A, _CHUNK), pad_a),
         jnp.pad(edge_index[1, _SPLIT:].reshape(_NW - _NTA, _NCH_B, _CHUNK),
                 pad_b)], axis=0)
    z128 = jnp.zeros((_ZR, _BN), f32)
    zdeg = jnp.zeros((_ZR, 16), f32)

    mesh = plsc.VectorSubcoreMesh(core_axis_name="c", subcore_axis_name="s",
                                  num_cores=_NC, num_subcores=_NS)
    deg_call = functools.partial(
        pl.kernel,
        out_type=[jax.ShapeDtypeStruct((_NC, _N, 16), f32)],
        mesh=mesh,
        scratch_types=[
            pltpu.VMEM((_PG, _CHUNK), jnp.int32),
            pltpu.VMEM((_CHUNK, 16), f32),
            pltpu.VMEM_SHARED((_SROWS, 16), f32),
            pltpu.SemaphoreType.DMA,
        ],
    )(_deg_body)
    (deg_p,) = deg_call(dst_h, zdeg)

    re_ = 2000
    c_arr = pl.pallas_call(
        _c_body,
        grid=(_E // re_,),
        in_specs=[pl.BlockSpec((re_, 16), hb),
                  _full((16, 128)), _full((1, 128))],
        out_specs=pl.BlockSpec((re_, 128), hb),
        out_shape=jax.ShapeDtypeStruct((_E, _BN), f32),
    )(edge_attr, wet, cvec)

    sc_call = functools.partial(
        pl.kernel,
        out_type=[jax.ShapeDtypeStruct((_NC, _N, _BN), f32)],
        mesh=mesh,
        scratch_types=[
            pltpu.VMEM((_PG, _CHUNK), jnp.int32),
            pltpu.VMEM((_PG, _CHUNK), jnp.int32),
            pltpu.VMEM((_PG, _CHUNK), jnp.int32),
            pltpu.VMEM((_CHUNK, _BN), f32),
            pltpu.VMEM((_CHUNK, _BN), f32),
            pltpu.VMEM((_CHUNK, _BN), f32),
            pltpu.VMEM_SHARED((_SROWS, _BN), f32),
            pltpu.SemaphoreType.DMA,
            pltpu.SemaphoreType.DMA,
            pltpu.SemaphoreType.DMA,
        ],
    )(_sc_body)
    (s_p,) = sc_call(src_h, dst_h, dst_h, a_arr, b_arr, c_arr, z128)


    out = pl.pallas_call(
        _out_body,
        grid=(_N // rn,),
        in_specs=[pl.BlockSpec((rn, 128), hb),
                  pl.BlockSpec((_NC, rn, 128), lambda i: (0, i, 0)),
                  pl.BlockSpec((_NC, rn, 16), lambda i: (0, i, 0)),
                  _full((128, 128)), _full((1, 128)),
                  _full((128, 64)), _full((1, 64)),
                  _full((128, 64)), _full((1, 64)),
                  _full((64, 64)), _full((1, 64)),
                  _full((64, 64)), _full((1, 64)),
                  _full((64, 128)), _full((1, 128))],
        out_specs=pl.BlockSpec((rn, 128), hb),
        out_shape=jax.ShapeDtypeStruct((_N, 128), f32),
    )(node_feat, s_p, deg_p, Wt2, bt22, Wpd, bpd2, Wpe, bpe2,
      Wp1, bp12, Wp2, bp22, Wp3, bp32)
    return out


# speedup vs baseline: 1.0014x; 1.0014x over previous
import functools

import jax
import jax.numpy as jnp
from jax import lax
from jax.experimental import pallas as pl
from jax.experimental.pallas import tpu as pltpu
from jax.experimental.pallas import tpu_sc as plsc

_N = 10000
_E = 320000
_BN = 128
_F32 = jnp.float32

_NC = 2
_NS = 16
_NW = _NC * _NS
_CHUNK = 64
_NCH_A = 156
_NCH_B = 157
_NTA = 24
_EPT_A = _NCH_A * _CHUNK
_EPT_B = _NCH_B * _CHUNK
_SPLIT = _NTA * _EPT_A
_NCHP = 160
_PG = 16
_SROWS = _N
_ZR = 8
_ZW = 79
_CP0 = 624


def _lk(x):
    return jnp.maximum(x, x * 0.01)


def _comb_body(ws, wd, we, be, bs, bd, wt1, bt1, wst_o, wdt_o, wet_o, cvec_o):
    wst_o[...] = jnp.dot(ws[...], wt1[...], preferred_element_type=_F32)
    wdt_o[...] = jnp.dot(wd[...], wt1[...], preferred_element_type=_F32)
    wet_o[...] = jnp.dot(we[...], wt1[...], preferred_element_type=_F32)
    bsum = be[...] + bs[...] + bd[...]
    cvec_o[...] = jnp.dot(bsum, wt1[...], preferred_element_type=_F32) + bt1[...]


def _ab_body(x, wst, wdt, a_o, b_o):
    xv = x[...]
    a_o[...] = jnp.dot(xv, wst[...], preferred_element_type=_F32)
    b_o[...] = jnp.dot(xv, wdt[...], preferred_element_type=_F32)


def _c_body(ea, wet, cvec, c_o):
    c_o[...] = jnp.dot(ea[...], wet[...], preferred_element_type=_F32) + cvec[...]


def _sc_body(src_h, dst_h, sidx_h, a_hbm, b_hbm, c_hbm, z128,
             s_out,
             src2d, dst2d, sidx2d, a_v, b_v, c_v, s_sh,
             sema, semb, semc):
    cid = lax.axis_index("c")
    sid = lax.axis_index("s")
    wid = sid * _NC + cid

    zrow = sid * (_N // _NS)

    def _zs(j, c):
        zo = jnp.minimum(zrow + j * _ZR, _N - _ZR)
        pltpu.sync_copy(z128, s_sh.at[pl.ds(zo, _ZR)])
        return c

    lax.fori_loop(0, _ZW, _zs, 0)
    plsc.subcore_barrier()

    ebase = jnp.where(wid < _NTA, wid * _EPT_A,
                      _SPLIT + (wid - _NTA) * _EPT_B)
    nch = jnp.where(wid < _NTA, _NCH_A, _NCH_B)

    def page(p, carry0):
        pbase = p * _PG

        def _ld(j, c):
            sl8 = pl.ds(j * 8, 8)
            slh = pl.ds(pbase + j * 8, 8)
            pltpu.sync_copy(src_h.at[wid, slh], src2d.at[sl8])
            pltpu.sync_copy(dst_h.at[wid, slh], dst2d.at[sl8])
            pltpu.sync_copy(sidx_h.at[wid, slh], sidx2d.at[sl8])
            return c

        lax.fori_loop(0, _PG // 8, _ld, 0)
        cnt = jnp.minimum(nch - pbase, _PG)

        def chunk(i, carry):
            it = pbase + i
            base = ebase + it * _CHUNK
            cpa = pltpu.async_copy(a_hbm.at[src2d.at[i]], a_v, sema)
            cpb = pltpu.async_copy(b_hbm.at[dst2d.at[i]], b_v, semb)
            cpc = pltpu.async_copy(c_hbm.at[pl.ds(base, _CHUNK)], c_v, semc)
            cpa.wait()
            cpb.wait()
            cpc.wait()

            @plsc.parallel_loop(0, _CHUNK, 1, unroll=4)
            def _row(r):
                for k in range(_BN // 16):
                    sl = pl.ds(k * 16, 16)
                    z = a_v[r, sl] + b_v[r, sl] + c_v[r, sl]
                    a_v[r, sl] = jnp.maximum(z, z * 0.01)
            pltpu.sync_copy(a_v, s_sh.at[sidx2d.at[i]], add=True)
            return carry

        lax.fori_loop(0, cnt, chunk, 0)
        return carry0

    lax.fori_loop(0, _NCHP // _PG, page, 0)
    plsc.subcore_barrier()

    off = sid * _CP0

    @pl.when(sid < _NS - 1)
    def _copy_main():
        pltpu.sync_copy(s_sh.at[pl.ds(off, _CP0)],
                        s_out.at[cid, pl.ds(off, _CP0)])

    @pl.when(sid == _NS - 1)
    def _copy_last():
        lo = (_NS - 1) * _CP0
        cp1 = _N - lo
        pltpu.sync_copy(s_sh.at[pl.ds(lo, cp1)], s_out.at[cid, pl.ds(lo, cp1)])


def _deg_body(dst_h, zdeg, deg_out,
              dst2d, ones_v, deg_sh, semd):
    cid = lax.axis_index("c")
    sid = lax.axis_index("s")
    wid = sid * _NC + cid

    def _ones_row(r, c):
        ones_v[r, :] = jnp.full((16,), 1.0, _F32)
        return c

    lax.fori_loop(0, _CHUNK, _ones_row, 0)

    zrow = sid * (_N // _NS)

    def _zs(j, c):
        zo = jnp.minimum(zrow + j * _ZR, _N - _ZR)
        pltpu.sync_copy(zdeg, deg_sh.at[pl.ds(zo, _ZR)])
        return c

    lax.fori_loop(0, _ZW, _zs, 0)
    plsc.subcore_barrier()

    nch = jnp.where(wid < _NTA, _NCH_A, _NCH_B)

    def page(p, carry0):
        pbase = p * _PG

        def _ld(j, c):
            pltpu.sync_copy(dst_h.at[wid, pl.ds(pbase + j * 8, 8)],
                            dst2d.at[pl.ds(j * 8, 8)])
            return c

        lax.fori_loop(0, _PG // 8, _ld, 0)
        cnt = jnp.minimum(nch - pbase, _PG)

        def chunk(i, carry):
            pltpu.sync_copy(ones_v, deg_sh.at[dst2d.at[i]], add=True)
            return carry

        lax.fori_loop(0, cnt, chunk, 0)
        return carry0

    lax.fori_loop(0, _NCHP // _PG, page, 0)
    plsc.subcore_barrier()

    off = sid * _CP0

    @pl.when(sid < _NS - 1)
    def _copy_main():
        pltpu.sync_copy(deg_sh.at[pl.ds(off, _CP0)],
                        deg_out.at[cid, pl.ds(off, _CP0)])

    @pl.when(sid == _NS - 1)
    def _copy_last():
        lo = (_NS - 1) * _CP0
        cp1 = _N - lo
        pltpu.sync_copy(deg_sh.at[pl.ds(lo, cp1)],
                        deg_out.at[cid, pl.ds(lo, cp1)])


def _out_body(x, sp, dp, wt2, bt2, wpd, bpd, wpe, bpe,
              wp1, bp1, wp2, bp2, wp3, bp3, o):
    s = sp[0] + sp[1]
    dsum = dp[0] + dp[1]
    deg = dsum[:, 0:1]
    msg = jnp.dot(s, wt2[...], preferred_element_type=_F32) + deg * bt2[...]
    h = (jnp.dot(x[...], wpd[...], preferred_element_type=_F32) + bpd[...]
         + jnp.dot(msg, wpe[...], preferred_element_type=_F32) + bpe[...])
    h = _lk(h)
    h = _lk(jnp.dot(h, wp1[...], preferred_element_type=_F32) + bp1[...])
    h = _lk(jnp.dot(h, wp2[...], preferred_element_type=_F32) + bp2[...])
    o[...] = jnp.dot(h, wp3[...], preferred_element_type=_F32) + bp3[...]


def _full(shape):
    return pl.BlockSpec(shape, lambda i: tuple(0 for _ in shape))


def kernel(node_feat, edge_index, edge_attr, We, be, Ws, bs, Wd, bd,
           Wt1, bt1, Wt2, bt2, Wpd, bpd, Wpe, bpe,
           Wp1, bp1, Wp2, bp2, Wp3, bp3):
    f32 = _F32
    be2, bs2, bd2, bt12 = (b.reshape(1, -1) for b in (be, bs, bd, bt1))
    bt22, bpd2, bpe2 = (b.reshape(1, -1) for b in (bt2, bpd, bpe))
    bp12, bp22, bp32 = (b.reshape(1, -1) for b in (bp1, bp2, bp3))

    wst, wdt, wet, cvec = pl.pallas_call(
        _comb_body,
        out_shape=[jax.ShapeDtypeStruct((128, 128), f32),
                   jax.ShapeDtypeStruct((128, 128), f32),
                   jax.ShapeDtypeStruct((16, 128), f32),
                   jax.ShapeDtypeStruct((1, 128), f32)],
    )(Ws, Wd, We, be2, bs2, bd2, Wt1, bt12)

    rn = 1000
    hb = lambda i: (i, 0)
    a_arr, b_arr = pl.pallas_call(
        _ab_body,
        grid=(_N // rn,),
        in_specs=[pl.BlockSpec((rn, 128), hb),
                  _full((128, 128)), _full((128, 128))],
        out_specs=[pl.BlockSpec((rn, 128), hb)] * 2,
        out_shape=[jax.ShapeDtypeStruct((_N, _BN), f32)] * 2,
    )(node_feat, wst, wdt)

    pad_a = ((0, 0), (0, _NCHP - _NCH_A), (0, 0))
    pad_b = ((0, 0), (0, _NCHP - _NCH_B), (0, 0))
    src_h = jnp.concatenate(
        [jnp.pad(edge_index[0, :_SPLIT].reshape(_NTA, _NCH_A, _CHUNK), pad_a),
         jnp.pad(edge_index[0, _SPLIT:].reshape(_NW - _NTA, _NCH_B, _CHUNK),
                 pad_b)], axis=0)
    dst_h = jnp.concatenate(
        [jnp.pad(edge_index[1, :_SPLIT].reshape(_NTA, _NCH_A, _CHUNK), pad_a),
         jnp.pad(edge_index[1, _SPLIT:].reshape(_NW - _NTA, _NCH_B, _CHUNK),
                 pad_b)], axis=0)
    z128 = jnp.zeros((_ZR, _BN), f32)
    zdeg = jnp.zeros((_ZR, 16), f32)

    mesh = plsc.VectorSubcoreMesh(core_axis_name="c", subcore_axis_name="s",
                                  num_cores=_NC, num_subcores=_NS)
    deg_call = functools.partial(
        pl.kernel,
        out_type=[jax.ShapeDtypeStruct((_NC, _N, 16), f32)],
        mesh=mesh,
        scratch_types=[
            pltpu.VMEM((_PG, _CHUNK), jnp.int32),
            pltpu.VMEM((_CHUNK, 16), f32),
            pltpu.VMEM_SHARED((_SROWS, 16), f32),
            pltpu.SemaphoreType.DMA,
        ],
    )(_deg_body)
    (deg_p,) = deg_call(dst_h, zdeg)

    re_ = 2000
    c_arr = pl.pallas_call(
        _c_body,
        grid=(_E // re_,),
        in_specs=[pl.BlockSpec((re_, 16), hb),
                  _full((16, 128)), _full((1, 128))],
        out_specs=pl.BlockSpec((re_, 128), hb),
        out_shape=jax.ShapeDtypeStruct((_E, _BN), f32),
    )(edge_attr, wet, cvec)

    sc_call = functools.partial(
        pl.kernel,
        out_type=[jax.ShapeDtypeStruct((_NC, _N, _BN), f32)],
        mesh=mesh,
        scratch_types=[
            pltpu.VMEM((_PG, _CHUNK), jnp.int32),
            pltpu.VMEM((_PG, _CHUNK), jnp.int32),
            pltpu.VMEM((_PG, _CHUNK), jnp.int32),
            pltpu.VMEM((_CHUNK, _BN), f32),
            pltpu.VMEM((_CHUNK, _BN), f32),
            pltpu.VMEM((_CHUNK, _BN), f32),
            pltpu.VMEM_SHARED((_SROWS, _BN), f32),
            pltpu.SemaphoreType.DMA,
            pltpu.SemaphoreType.DMA,
            pltpu.SemaphoreType.DMA,
        ],
    )(_sc_body)
    (s_p,) = sc_call(src_h, dst_h, dst_h, a_arr, b_arr, c_arr, z128)


    out = pl.pallas_call(
        _out_body,
        grid=(_N // rn,),
        in_specs=[pl.BlockSpec((rn, 128), hb),
                  pl.BlockSpec((_NC, rn, 128), lambda i: (0, i, 0)),
                  pl.BlockSpec((_NC, rn, 16), lambda i: (0, i, 0)),
                  _full((128, 128)), _full((1, 128)),
                  _full((128, 64)), _full((1, 64)),
                  _full((128, 64)), _full((1, 64)),
                  _full((64, 64)), _full((1, 64)),
                  _full((64, 64)), _full((1, 64)),
                  _full((64, 128)), _full((1, 128))],
        out_specs=pl.BlockSpec((rn, 128), hb),
        out_shape=jax.ShapeDtypeStruct((_N, 128), f32),
    )(node_feat, s_p, deg_p, Wt2, bt22, Wpd, bpd2, Wpe, bpe2,
      Wp1, bp12, Wp2, bp22, Wp3, bp32)
    return out


# speedup vs baseline: 1.0102x; 1.0088x over previous
import functools

import jax
import jax.numpy as jnp
from jax import lax
from jax.experimental import pallas as pl
from jax.experimental.pallas import tpu as pltpu
from jax.experimental.pallas import tpu_sc as plsc

_N = 10000
_E = 320000
_BN = 128
_F32 = jnp.float32

_NC = 2
_NS = 16
_NW = _NC * _NS
_CHUNK = 64
_NCH_A = 156
_NCH_B = 157
_NTA = 24
_EPT_A = _NCH_A * _CHUNK
_EPT_B = _NCH_B * _CHUNK
_SPLIT = _NTA * _EPT_A
_NCHP = 160
_PG = 16
_SROWS = _N
_ZR = 8
_ZW = 79
_CP0 = 624


def _lk(x):
    return jnp.maximum(x, x * 0.01)


def _comb_body(ws, wd, we, be, bs, bd, wt1, bt1, wst_o, wdt_o, wet_o, cvec_o):
    wst_o[...] = jnp.dot(ws[...], wt1[...], preferred_element_type=_F32)
    wdt_o[...] = jnp.dot(wd[...], wt1[...], preferred_element_type=_F32)
    wet_o[...] = jnp.dot(we[...], wt1[...], preferred_element_type=_F32)
    bsum = be[...] + bs[...] + bd[...]
    cvec_o[...] = jnp.dot(bsum, wt1[...], preferred_element_type=_F32) + bt1[...]


def _ab_body(x, wst, wdt, a_o, b_o):
    xv = x[...]
    a_o[...] = jnp.dot(xv, wst[...], preferred_element_type=_F32)
    b_o[...] = jnp.dot(xv, wdt[...], preferred_element_type=_F32)


def _c_body(ea, wet, cvec, c_o):
    c_o[...] = jnp.dot(ea[...], wet[...], preferred_element_type=_F32) + cvec[...]


def _sc_body(src_h, dst_h, sidx_h, a_hbm, b_hbm, c_hbm, z128,
             s_out,
             src2d, dst2d, sidx2d, a_v, b_v, c_v, s_sh,
             sema, semb, semc):
    cid = lax.axis_index("c")
    sid = lax.axis_index("s")
    wid = sid * _NC + cid

    zrow = sid * (_N // _NS)

    def _zs(j, c):
        zo = jnp.minimum(zrow + j * _ZR, _N - _ZR)
        pltpu.sync_copy(z128, s_sh.at[pl.ds(zo, _ZR)])
        return c

    lax.fori_loop(0, _ZW, _zs, 0)
    plsc.subcore_barrier()

    ebase = jnp.where(wid < _NTA, wid * _EPT_A,
                      _SPLIT + (wid - _NTA) * _EPT_B)
    nch = jnp.where(wid < _NTA, _NCH_A, _NCH_B)

    def page(p, carry0):
        pbase = p * _PG

        def _ld(j, c):
            sl8 = pl.ds(j * 8, 8)
            slh = pl.ds(pbase + j * 8, 8)
            pltpu.sync_copy(src_h.at[wid, slh], src2d.at[sl8])
            pltpu.sync_copy(dst_h.at[wid, slh], dst2d.at[sl8])
            pltpu.sync_copy(sidx_h.at[wid, slh], sidx2d.at[sl8])
            return c

        lax.fori_loop(0, _PG // 8, _ld, 0)
        cnt = jnp.minimum(nch - pbase, _PG)

        def chunk(i, carry):
            it = pbase + i
            base = ebase + it * _CHUNK
            cpa = pltpu.async_copy(a_hbm.at[src2d.at[i]], a_v, sema)
            cpb = pltpu.async_copy(b_hbm.at[dst2d.at[i]], b_v, semb)
            cpc = pltpu.async_copy(c_hbm.at[pl.ds(base, _CHUNK)], c_v, semc)
            cpa.wait()
            cpb.wait()
            cpc.wait()

            @plsc.parallel_loop(0, _CHUNK, 1, unroll=4)
            def _row(r):
                for k in range(_BN // 16):
                    sl = pl.ds(k * 16, 16)
                    z = a_v[r, sl] + b_v[r, sl] + c_v[r, sl]
                    a_v[r, sl] = jnp.maximum(z, z * 0.01)
            pltpu.sync_copy(a_v, s_sh.at[sidx2d.at[i]], add=True)
            return carry

        lax.fori_loop(0, cnt, chunk, 0)
        return carry0

    lax.fori_loop(0, _NCHP // _PG, page, 0)
    plsc.subcore_barrier()

    off = sid * _CP0

    @pl.when(sid < _NS - 1)
    def _copy_main():
        pltpu.sync_copy(s_sh.at[pl.ds(off, _CP0)],
                        s_out.at[cid, pl.ds(off, _CP0)])

    @pl.when(sid == _NS - 1)
    def _copy_last():
        lo = (_NS - 1) * _CP0
        cp1 = _N - lo
        pltpu.sync_copy(s_sh.at[pl.ds(lo, cp1)], s_out.at[cid, pl.ds(lo, cp1)])


def _deg_body(dst_h, zdeg, deg_out,
              dst2d, ones_v, deg_sh, semd):
    cid = lax.axis_index("c")
    sid = lax.axis_index("s")
    wid = sid * _NC + cid

    def _ones_row(r, c):
        ones_v[r, :] = jnp.full((16,), 1.0, _F32)
        return c

    lax.fori_loop(0, _CHUNK, _ones_row, 0)

    zrow = sid * (_N // _NS)

    def _zs(j, c):
        zo = jnp.minimum(zrow + j * _ZR, _N - _ZR)
        pltpu.sync_copy(zdeg, deg_sh.at[pl.ds(zo, _ZR)])
        return c

    lax.fori_loop(0, _ZW, _zs, 0)
    plsc.subcore_barrier()

    nch = jnp.where(wid < _NTA, _NCH_A, _NCH_B)

    def page(p, carry0):
        pbase = p * _PG

        def _ld(j, c):
            pltpu.sync_copy(dst_h.at[wid, pl.ds(pbase + j * 8, 8)],
                            dst2d.at[pl.ds(j * 8, 8)])
            return c

        lax.fori_loop(0, _PG // 8, _ld, 0)
        cnt = jnp.minimum(nch - pbase, _PG)

        def chunk(i, carry):
            pltpu.async_copy(ones_v, deg_sh.at[dst2d.at[i]], semd, add=True)
            return carry

        lax.fori_loop(0, cnt, chunk, 0)

        def drain(i, carry):
            pltpu.make_async_copy(ones_v, deg_sh.at[dst2d.at[i]], semd).wait()
            return carry

        lax.fori_loop(0, cnt, drain, 0)
        return carry0

    lax.fori_loop(0, _NCHP // _PG, page, 0)
    plsc.subcore_barrier()

    off = sid * _CP0

    @pl.when(sid < _NS - 1)
    def _copy_main():
        pltpu.sync_copy(deg_sh.at[pl.ds(off, _CP0)],
                        deg_out.at[cid, pl.ds(off, _CP0)])

    @pl.when(sid == _NS - 1)
    def _copy_last():
        lo = (_NS - 1) * _CP0
        cp1 = _N - lo
        pltpu.sync_copy(deg_sh.at[pl.ds(lo, cp1)],
                        deg_out.at[cid, pl.ds(lo, cp1)])


def _out_body(x, sp, dp, wt2, bt2, wpd, bpd, wpe, bpe,
              wp1, bp1, wp2, bp2, wp3, bp3, o):
    s = sp[0] + sp[1]
    dsum = dp[0] + dp[1]
    deg = dsum[:, 0:1]
    msg = jnp.dot(s, wt2[...], preferred_element_type=_F32) + deg * bt2[...]
    h = (jnp.dot(x[...], wpd[...], preferred_element_type=_F32) + bpd[...]
         + jnp.dot(msg, wpe[...], preferred_element_type=_F32) + bpe[...])
    h = _lk(h)
    h = _lk(jnp.dot(h, wp1[...], preferred_element_type=_F32) + bp1[...])
    h = _lk(jnp.dot(h, wp2[...], preferred_element_type=_F32) + bp2[...])
    o[...] = jnp.dot(h, wp3[...], preferred_element_type=_F32) + bp3[...]


def _full(shape):
    return pl.BlockSpec(shape, lambda i: tuple(0 for _ in shape))


def kernel(node_feat, edge_index, edge_attr, We, be, Ws, bs, Wd, bd,
           Wt1, bt1, Wt2, bt2, Wpd, bpd, Wpe, bpe,
           Wp1, bp1, Wp2, bp2, Wp3, bp3):
    f32 = _F32
    be2, bs2, bd2, bt12 = (b.reshape(1, -1) for b in (be, bs, bd, bt1))
    bt22, bpd2, bpe2 = (b.reshape(1, -1) for b in (bt2, bpd, bpe))
    bp12, bp22, bp32 = (b.reshape(1, -1) for b in (bp1, bp2, bp3))

    wst, wdt, wet, cvec = pl.pallas_call(
        _comb_body,
        out_shape=[jax.ShapeDtypeStruct((128, 128), f32),
                   jax.ShapeDtypeStruct((128, 128), f32),
                   jax.ShapeDtypeStruct((16, 128), f32),
                   jax.ShapeDtypeStruct((1, 128), f32)],
    )(Ws, Wd, We, be2, bs2, bd2, Wt1, bt12)

    rn = 1000
    hb = lambda i: (i, 0)
    a_arr, b_arr = pl.pallas_call(
        _ab_body,
        grid=(_N // rn,),
        in_specs=[pl.BlockSpec((rn, 128), hb),
                  _full((128, 128)), _full((128, 128))],
        out_specs=[pl.BlockSpec((rn, 128), hb)] * 2,
        out_shape=[jax.ShapeDtypeStruct((_N, _BN), f32)] * 2,
    )(node_feat, wst, wdt)

    pad_a = ((0, 0), (0, _NCHP - _NCH_A), (0, 0))
    pad_b = ((0, 0), (0, _NCHP - _NCH_B), (0, 0))
    src_h = jnp.concatenate(
        [jnp.pad(edge_index[0, :_SPLIT].reshape(_NTA, _NCH_A, _CHUNK), pad_a),
         jnp.pad(edge_index[0, _SPLIT:].reshape(_NW - _NTA, _NCH_B, _CHUNK),
                 pad_b)], axis=0)
    dst_h = jnp.concatenate(
        [jnp.pad(edge_index[1, :_SPLIT].reshape(_NTA, _NCH_A, _CHUNK), pad_a),
         jnp.pad(edge_index[1, _SPLIT:].reshape(_NW - _NTA, _NCH_B, _CHUNK),
                 pad_b)], axis=0)
    z128 = jnp.zeros((_ZR, _BN), f32)
    zdeg = jnp.zeros((_ZR, 16), f32)

    mesh = plsc.VectorSubcoreMesh(core_axis_name="c", subcore_axis_name="s",
                                  num_cores=_NC, num_subcores=_NS)
    deg_call = functools.partial(
        pl.kernel,
        out_type=[jax.ShapeDtypeStruct((_NC, _N, 16), f32)],
        mesh=mesh,
        scratch_types=[
            pltpu.VMEM((_PG, _CHUNK), jnp.int32),
            pltpu.VMEM((_CHUNK, 16), f32),
            pltpu.VMEM_SHARED((_SROWS, 16), f32),
            pltpu.SemaphoreType.DMA,
        ],
    )(_deg_body)
    (deg_p,) = deg_call(dst_h, zdeg)

    re_ = 2000
    c_arr = pl.pallas_call(
        _c_body,
        grid=(_E // re_,),
        in_specs=[pl.BlockSpec((re_, 16), hb),
                  _full((16, 128)), _full((1, 128))],
        out_specs=pl.BlockSpec((re_, 128), hb),
        out_shape=jax.ShapeDtypeStruct((_E, _BN), f32),
    )(edge_attr, wet, cvec)

    sc_call = functools.partial(
        pl.kernel,
        out_type=[jax.ShapeDtypeStruct((_NC, _N, _BN), f32)],
        mesh=mesh,
        scratch_types=[
            pltpu.VMEM((_PG, _CHUNK), jnp.int32),
            pltpu.VMEM((_PG, _CHUNK), jnp.int32),
            pltpu.VMEM((_PG, _CHUNK), jnp.int32),
            pltpu.VMEM((_CHUNK, _BN), f32),
            pltpu.VMEM((_CHUNK, _BN), f32),
            pltpu.VMEM((_CHUNK, _BN), f32),
            pltpu.VMEM_SHARED((_SROWS, _BN), f32),
            pltpu.SemaphoreType.DMA,
            pltpu.SemaphoreType.DMA,
            pltpu.SemaphoreType.DMA,
        ],
    )(_sc_body)
    (s_p,) = sc_call(src_h, dst_h, dst_h, a_arr, b_arr, c_arr, z128)


    out = pl.pallas_call(
        _out_body,
        grid=(_N // rn,),
        in_specs=[pl.BlockSpec((rn, 128), hb),
                  pl.BlockSpec((_NC, rn, 128), lambda i: (0, i, 0)),
                  pl.BlockSpec((_NC, rn, 16), lambda i: (0, i, 0)),
                  _full((128, 128)), _full((1, 128)),
                  _full((128, 64)), _full((1, 64)),
                  _full((128, 64)), _full((1, 64)),
                  _full((64, 64)), _full((1, 64)),
                  _full((64, 64)), _full((1, 64)),
                  _full((64, 128)), _full((1, 128))],
        out_specs=pl.BlockSpec((rn, 128), hb),
        out_shape=jax.ShapeDtypeStruct((_N, 128), f32),
    )(node_feat, s_p, deg_p, Wt2, bt22, Wpd, bpd2, Wpe, bpe2,
      Wp1, bp12, Wp2, bp22, Wp3, bp32)
    return out
